# baseline (speedup 1.0000x reference)
import functools
from math import sqrt

import numpy as np
import jax
import jax.numpy as jnp
from jax import lax
from jax.experimental import pallas as pl
from jax.experimental.pallas import tpu as pltpu

EPS = 1e-5


def _qkv_kernel(x_ref, w_ref, b_ref, a_ref, g_ref, be_ref,
                m_ref, mt_ref, ic_ref, o_ref, *, T, F):
    Ctot = w_ref.shape[-1]
    xT = jnp.transpose(x_ref[0].astype(jnp.bfloat16))

    y = jnp.dot(xT, w_ref[...], preferred_element_type=jnp.float32)
    y = y + b_ref[...]
    y = jnp.where(y >= 0.0, y, a_ref[...] * y)
    y3 = y.reshape(T, F, Ctot)

    s1 = jnp.sum(y3, axis=1)
    mu_g = jnp.dot(s1, m_ref[...],
                   preferred_element_type=jnp.float32) * ic_ref[...]
    mu = jnp.dot(mu_g, mt_ref[...], preferred_element_type=jnp.float32)
    d = y3 - mu[:, None, :]
    s2 = jnp.sum(d * d, axis=1)
    var_g = jnp.dot(s2, m_ref[...],
                    preferred_element_type=jnp.float32) * ic_ref[...]
    inv = jnp.dot(lax.rsqrt(var_g + EPS), mt_ref[...],
                  preferred_element_type=jnp.float32)
    z = d * inv[:, None, :] * g_ref[...][None] + be_ref[...][None]

    o_ref[0] = jnp.transpose(z.astype(jnp.bfloat16), (0, 2, 1))


def _attn_proj_kernel(z_ref, x_ref, wp_ref, bp_ref, ap_ref, gp_ref,
                      bep_ref, o_ref, *, H, E, Dh, T, F, scale):
    D = x_ref.shape[1]
    P = T * F
    EF = E * F
    DhF = Dh * F
    zz = z_ref[0]
    qoff, koff, voff = 0, H * EF, 2 * H * EF

    a_parts = []
    for h in range(H):
        qh = zz[:, qoff + h * EF:qoff + (h + 1) * EF]
        kh = zz[:, koff + h * EF:koff + (h + 1) * EF]
        vh = zz[:, voff + h * DhF:voff + (h + 1) * DhF]
        s = lax.dot_general(qh, kh, (((1,), (1,)), ((), ())),
                            preferred_element_type=jnp.float32) * scale
        mx = jnp.max(s, axis=-1, keepdims=True)
        p = jnp.exp(s - mx)
        p = p * (1.0 / jnp.sum(p, axis=-1, keepdims=True))
        ah = jnp.dot(p.astype(jnp.bfloat16), vh,
                     preferred_element_type=jnp.float32)
        a_parts.append(ah.astype(jnp.bfloat16))
    Aall = jnp.concatenate(a_parts, axis=-1)
    At = jnp.transpose(Aall.reshape(T, D, F), (0, 2, 1))
    A2 = At.reshape(P, D)

    o = jnp.dot(A2, wp_ref[...], preferred_element_type=jnp.float32)
    o = o + bp_ref[...]
    o = jnp.where(o >= 0.0, o, ap_ref[...] * o)
    o3 = o.reshape(T, F, D)
    mu2 = jnp.mean(o3, axis=(1, 2), keepdims=True)
    d2 = o3 - mu2
    var2 = jnp.mean(d2 * d2, axis=(1, 2), keepdims=True)
    on = d2 * lax.rsqrt(var2 + EPS) * gp_ref[...][None] + bep_ref[...][None]

    o_ref[0] = jnp.transpose(on.reshape(P, D)) + x_ref[0]


def _pack(W, bias, alpha, gamma, beta):
    G, Cin, Cout = W.shape
    F = gamma.shape[1]
    Wc = jnp.transpose(W, (1, 0, 2)).reshape(Cin, G * Cout)
    bc = jnp.transpose(bias, (1, 0, 2)).reshape(1, G * Cout)
    ac = jnp.repeat(alpha.reshape(G, 1), Cout, axis=1).reshape(1, G * Cout)
    gc = jnp.transpose(gamma, (1, 0, 2)).reshape(F, G * Cout)
    bec = jnp.transpose(beta, (1, 0, 2)).reshape(F, G * Cout)
    return Wc, bc, ac, gc, bec


def kernel(x, q_W, q_bias, q_alpha, q_gamma, q_beta,
           k_W, k_bias, k_alpha, k_gamma, k_beta,
           v_W, v_bias, v_alpha, v_gamma, v_beta,
           proj_W, proj_bias, proj_alpha, proj_gamma, proj_beta):
    B, D, T, F = x.shape
    H, _, E = q_W.shape
    Dh = D // H
    P = T * F
    Ctot = 2 * H * E + H * Dh
    NG = 3 * H

    pq = _pack(q_W, q_bias, q_alpha, q_gamma, q_beta)
    pk = _pack(k_W, k_bias, k_alpha, k_gamma, k_beta)
    pv = _pack(v_W, v_bias, v_alpha, v_gamma, v_beta)
    W_cat, b_cat, a_cat, g_cat, be_cat = (
        jnp.concatenate([pq[i], pk[i], pv[i]], axis=1) for i in range(5))

    sizes = [E] * H + [E] * H + [Dh] * H
    gid = np.repeat(np.arange(NG), sizes)
    M = jnp.asarray((gid[:, None] == np.arange(NG)[None, :]).astype(np.float32))
    Mt = M.T
    invcnt = jnp.asarray(1.0 / (F * np.asarray(sizes, np.float32)))[None, :]

    x2 = x.reshape(B, D, P)

    kern_a = functools.partial(_qkv_kernel, T=T, F=F)
    zs = pl.pallas_call(
        kern_a,
        out_shape=jax.ShapeDtypeStruct((B, T, Ctot, F), jnp.bfloat16),
        grid=(B,),
        in_specs=[
            pl.BlockSpec((1, D, P), lambda b: (b, 0, 0)),
            pl.BlockSpec((D, Ctot), lambda b: (0, 0)),
            pl.BlockSpec((1, Ctot), lambda b: (0, 0)),
            pl.BlockSpec((1, Ctot), lambda b: (0, 0)),
            pl.BlockSpec((F, Ctot), lambda b: (0, 0)),
            pl.BlockSpec((F, Ctot), lambda b: (0, 0)),
            pl.BlockSpec((Ctot, NG), lambda b: (0, 0)),
            pl.BlockSpec((NG, Ctot), lambda b: (0, 0)),
            pl.BlockSpec((1, NG), lambda b: (0, 0)),
        ],
        out_specs=pl.BlockSpec((1, T, Ctot, F), lambda b: (b, 0, 0, 0)),
        compiler_params=pltpu.CompilerParams(
            dimension_semantics=("parallel",),
            vmem_limit_bytes=100 * 1024 * 1024),
    )(x2, W_cat.astype(jnp.bfloat16), b_cat, a_cat, g_cat, be_cat,
      M, Mt, invcnt)

    z_flat = zs.reshape(B, T, Ctot * F)

    kern_b = functools.partial(_attn_proj_kernel, H=H, E=E, Dh=Dh, T=T, F=F,
                               scale=1.0 / sqrt(F * E))
    out = pl.pallas_call(
        kern_b,
        out_shape=jax.ShapeDtypeStruct((B, D, P), jnp.float32),
        grid=(B,),
        in_specs=[
            pl.BlockSpec((1, T, Ctot * F), lambda b: (b, 0, 0)),
            pl.BlockSpec((1, D, P), lambda b: (b, 0, 0)),
            pl.BlockSpec((D, D), lambda b: (0, 0)),
            pl.BlockSpec((1, D), lambda b: (0, 0)),
            pl.BlockSpec((1, D), lambda b: (0, 0)),
            pl.BlockSpec((F, D), lambda b: (0, 0)),
            pl.BlockSpec((F, D), lambda b: (0, 0)),
        ],
        out_specs=pl.BlockSpec((1, D, P), lambda b: (b, 0, 0)),
        compiler_params=pltpu.CompilerParams(
            dimension_semantics=("parallel",),
            vmem_limit_bytes=100 * 1024 * 1024),
    )(z_flat, x2, proj_W[0].astype(jnp.bfloat16), proj_bias[0],
      jnp.broadcast_to(proj_alpha[0].reshape(1, 1), (1, D)),
      proj_gamma[0], proj_beta[0])

    return out.reshape(B, D, T, F)


# speedup vs baseline: 1.5943x; 1.5943x over previous
import functools
from math import sqrt

import numpy as np
import jax
import jax.numpy as jnp
from jax import lax
from jax.experimental import pallas as pl
from jax.experimental.pallas import tpu as pltpu

EPS = 1e-5


def _fused_kernel(x_ref, w_ref, b_ref, a_ref, g_ref, be_ref,
                  m_ref, mt_ref, ic_ref,
                  wp_ref, bp_ref, ap_ref, gp_ref, bep_ref,
                  o_ref, *, H, E, Dh, T, F, scale):
    D = x_ref.shape[1]
    P = T * F
    EF = E * F
    DhF = Dh * F
    Ctot = w_ref.shape[-1]

    xf = x_ref[0]
    xT = jnp.transpose(xf.astype(jnp.bfloat16))

    y = jnp.dot(xT, w_ref[...], preferred_element_type=jnp.float32)
    y = y + b_ref[...]
    y = jnp.where(y >= 0.0, y, a_ref[...] * y)
    y3 = y.reshape(T, F, Ctot)

    s1 = jnp.sum(y3, axis=1)
    mu_g = jnp.dot(s1, m_ref[...],
                   preferred_element_type=jnp.float32) * ic_ref[...]
    mu = jnp.dot(mu_g, mt_ref[...], preferred_element_type=jnp.float32)
    d = y3 - mu[:, None, :]
    s2 = jnp.sum(d * d, axis=1)
    var_g = jnp.dot(s2, m_ref[...],
                    preferred_element_type=jnp.float32) * ic_ref[...]
    inv = jnp.dot(lax.rsqrt(var_g + EPS), mt_ref[...],
                  preferred_element_type=jnp.float32)
    z = d * inv[:, None, :] * g_ref[...][None] + be_ref[...][None]

    zs = jnp.transpose(z.astype(jnp.bfloat16), (0, 2, 1))
    zz = zs.reshape(T, Ctot * F)
    qoff, koff, voff = 0, H * EF, 2 * H * EF

    a_parts = []
    for h in range(H):
        qh = zz[:, qoff + h * EF:qoff + (h + 1) * EF]
        kh = zz[:, koff + h * EF:koff + (h + 1) * EF]
        vh = zz[:, voff + h * DhF:voff + (h + 1) * DhF]
        s = lax.dot_general(qh, kh, (((1,), (1,)), ((), ())),
                            preferred_element_type=jnp.float32) * scale
        mx = jnp.max(s, axis=-1, keepdims=True)
        p = jnp.exp(s - mx)
        p = p * (1.0 / jnp.sum(p, axis=-1, keepdims=True))
        ah = jnp.dot(p.astype(jnp.bfloat16), vh,
                     preferred_element_type=jnp.float32)
        a_parts.append(ah.astype(jnp.bfloat16))
    Aall = jnp.concatenate(a_parts, axis=-1)
    At = jnp.transpose(Aall.reshape(T, D, F), (0, 2, 1))
    A2 = At.reshape(P, D)

    o = jnp.dot(A2, wp_ref[...], preferred_element_type=jnp.float32)
    o = o + bp_ref[...]
    o = jnp.where(o >= 0.0, o, ap_ref[...] * o)
    o3 = o.reshape(T, F, D)
    mu2 = jnp.mean(o3, axis=(1, 2), keepdims=True)
    d2 = o3 - mu2
    var2 = jnp.mean(d2 * d2, axis=(1, 2), keepdims=True)
    on = d2 * lax.rsqrt(var2 + EPS) * gp_ref[...][None] + bep_ref[...][None]

    o_ref[0] = jnp.transpose(on.reshape(P, D)) + xf


def _pack(W, bias, alpha, gamma, beta):
    G, Cin, Cout = W.shape
    F = gamma.shape[1]
    Wc = jnp.transpose(W, (1, 0, 2)).reshape(Cin, G * Cout)
    bc = jnp.transpose(bias, (1, 0, 2)).reshape(1, G * Cout)
    ac = jnp.repeat(alpha.reshape(G, 1), Cout, axis=1).reshape(1, G * Cout)
    gc = jnp.transpose(gamma, (1, 0, 2)).reshape(F, G * Cout)
    bec = jnp.transpose(beta, (1, 0, 2)).reshape(F, G * Cout)
    return Wc, bc, ac, gc, bec


def kernel(x, q_W, q_bias, q_alpha, q_gamma, q_beta,
           k_W, k_bias, k_alpha, k_gamma, k_beta,
           v_W, v_bias, v_alpha, v_gamma, v_beta,
           proj_W, proj_bias, proj_alpha, proj_gamma, proj_beta):
    B, D, T, F = x.shape
    H, _, E = q_W.shape
    Dh = D // H
    P = T * F
    Ctot = 2 * H * E + H * Dh
    NG = 3 * H

    pq = _pack(q_W, q_bias, q_alpha, q_gamma, q_beta)
    pk = _pack(k_W, k_bias, k_alpha, k_gamma, k_beta)
    pv = _pack(v_W, v_bias, v_alpha, v_gamma, v_beta)
    W_cat, b_cat, a_cat, g_cat, be_cat = (
        jnp.concatenate([pq[i], pk[i], pv[i]], axis=1) for i in range(5))

    sizes = [E] * H + [E] * H + [Dh] * H
    gid = np.repeat(np.arange(NG), sizes)
    M = jnp.asarray((gid[:, None] == np.arange(NG)[None, :]).astype(np.float32))
    Mt = M.T
    invcnt = jnp.asarray(1.0 / (F * np.asarray(sizes, np.float32)))[None, :]

    x2 = x.reshape(B, D, P)
    kern = functools.partial(_fused_kernel, H=H, E=E, Dh=Dh, T=T, F=F,
                             scale=1.0 / sqrt(F * E))
    out = pl.pallas_call(
        kern,
        out_shape=jax.ShapeDtypeStruct((B, D, P), jnp.float32),
        grid=(B,),
        in_specs=[
            pl.BlockSpec((1, D, P), lambda b: (b, 0, 0)),
            pl.BlockSpec((D, Ctot), lambda b: (0, 0)),
            pl.BlockSpec((1, Ctot), lambda b: (0, 0)),
            pl.BlockSpec((1, Ctot), lambda b: (0, 0)),
            pl.BlockSpec((F, Ctot), lambda b: (0, 0)),
            pl.BlockSpec((F, Ctot), lambda b: (0, 0)),
            pl.BlockSpec((Ctot, NG), lambda b: (0, 0)),
            pl.BlockSpec((NG, Ctot), lambda b: (0, 0)),
            pl.BlockSpec((1, NG), lambda b: (0, 0)),
            pl.BlockSpec((D, D), lambda b: (0, 0)),
            pl.BlockSpec((1, D), lambda b: (0, 0)),
            pl.BlockSpec((1, D), lambda b: (0, 0)),
            pl.BlockSpec((F, D), lambda b: (0, 0)),
            pl.BlockSpec((F, D), lambda b: (0, 0)),
        ],
        out_specs=pl.BlockSpec((1, D, P), lambda b: (b, 0, 0)),
        compiler_params=pltpu.CompilerParams(
            dimension_semantics=("parallel",),
            vmem_limit_bytes=100 * 1024 * 1024),
    )(x2, W_cat.astype(jnp.bfloat16), b_cat, a_cat, g_cat, be_cat,
      M, Mt, invcnt,
      proj_W[0].astype(jnp.bfloat16), proj_bias[0],
      jnp.broadcast_to(proj_alpha[0].reshape(1, 1), (1, D)),
      proj_gamma[0], proj_beta[0])

    return out.reshape(B, D, T, F)


# speedup vs baseline: 1.7573x; 1.1022x over previous
import functools
from math import sqrt

import numpy as np
import jax
import jax.numpy as jnp
from jax import lax
from jax.experimental import pallas as pl
from jax.experimental.pallas import tpu as pltpu

EPS = 1e-5


def _fused_kernel(x_ref, w_ref, b_ref, a_ref, g_ref, be_ref,
                  m_ref, mt_ref, ic_ref, st_ref,
                  wp_ref, bp_ref, ap_ref, gp_ref, bep_ref,
                  o_ref, *, H, E, Dh, T, F, scale):
    D = x_ref.shape[1]
    P = T * F
    EF = E * F
    DhF = Dh * F
    Ctot = w_ref.shape[-1]

    xf = x_ref[0]
    xT = jnp.transpose(xf.astype(jnp.bfloat16))

    y = jnp.dot(xT, w_ref[...], preferred_element_type=jnp.float32)
    y = y + b_ref[...]
    y = jnp.where(y >= 0.0, y, a_ref[...] * y)
    y3 = y.reshape(T, F, Ctot)

    yb = y.astype(jnp.bfloat16)
    s1 = jnp.dot(st_ref[...], yb, preferred_element_type=jnp.float32)
    s2 = jnp.dot(st_ref[...], (yb * yb).astype(jnp.bfloat16),
                 preferred_element_type=jnp.float32)
    mu_g = jnp.dot(s1, m_ref[...],
                   preferred_element_type=jnp.float32) * ic_ref[...]
    sq_g = jnp.dot(s2, m_ref[...],
                   preferred_element_type=jnp.float32) * ic_ref[...]
    inv_g = lax.rsqrt(sq_g - mu_g * mu_g + EPS)
    mu = jnp.dot(mu_g, mt_ref[...], preferred_element_type=jnp.float32)
    inv = jnp.dot(inv_g, mt_ref[...], preferred_element_type=jnp.float32)
    z = (y3 - mu[:, None, :]) * inv[:, None, :] * g_ref[...][None] \
        + be_ref[...][None]

    zs = jnp.transpose(z.astype(jnp.bfloat16), (0, 2, 1))
    zz = zs.reshape(T, Ctot * F)
    qoff, koff, voff = 0, H * EF, 2 * H * EF

    ss = []
    for h in range(H):
        qh = zz[:, qoff + h * EF:qoff + (h + 1) * EF]
        kh = zz[:, koff + h * EF:koff + (h + 1) * EF]
        ss.append(lax.dot_general(qh, kh, (((1,), (1,)), ((), ())),
                                  preferred_element_type=jnp.float32) * scale)
    ps = []
    for h in range(H):
        s = ss[h]
        mx = jnp.max(s, axis=-1, keepdims=True)
        p = jnp.exp(s - mx)
        p = p * (1.0 / jnp.sum(p, axis=-1, keepdims=True))
        ps.append(p.astype(jnp.bfloat16))
    a_parts = []
    for h in range(H):
        vh = zz[:, voff + h * DhF:voff + (h + 1) * DhF]
        ah = jnp.dot(ps[h], vh, preferred_element_type=jnp.float32)
        a_parts.append(ah.astype(jnp.bfloat16))
    Aall = jnp.concatenate(a_parts, axis=-1)
    At = jnp.transpose(Aall.reshape(T, D, F), (0, 2, 1))
    A2 = At.reshape(P, D)

    o = jnp.dot(A2, wp_ref[...], preferred_element_type=jnp.float32)
    o = o + bp_ref[...]
    o = jnp.where(o >= 0.0, o, ap_ref[...] * o)
    o3 = o.reshape(T, F, D)
    mu2 = jnp.mean(o3, axis=(1, 2), keepdims=True)
    d2 = o3 - mu2
    var2 = jnp.mean(d2 * d2, axis=(1, 2), keepdims=True)
    on = d2 * lax.rsqrt(var2 + EPS) * gp_ref[...][None] + bep_ref[...][None]

    o_ref[0] = jnp.transpose(on.reshape(P, D)) + xf


def _pack(W, bias, alpha, gamma, beta):
    G, Cin, Cout = W.shape
    F = gamma.shape[1]
    Wc = jnp.transpose(W, (1, 0, 2)).reshape(Cin, G * Cout)
    bc = jnp.transpose(bias, (1, 0, 2)).reshape(1, G * Cout)
    ac = jnp.repeat(alpha.reshape(G, 1), Cout, axis=1).reshape(1, G * Cout)
    gc = jnp.transpose(gamma, (1, 0, 2)).reshape(F, G * Cout)
    bec = jnp.transpose(beta, (1, 0, 2)).reshape(F, G * Cout)
    return Wc, bc, ac, gc, bec


def kernel(x, q_W, q_bias, q_alpha, q_gamma, q_beta,
           k_W, k_bias, k_alpha, k_gamma, k_beta,
           v_W, v_bias, v_alpha, v_gamma, v_beta,
           proj_W, proj_bias, proj_alpha, proj_gamma, proj_beta):
    B, D, T, F = x.shape
    H, _, E = q_W.shape
    Dh = D // H
    P = T * F
    Ctot = 2 * H * E + H * Dh
    NG = 3 * H

    pq = _pack(q_W, q_bias, q_alpha, q_gamma, q_beta)
    pk = _pack(k_W, k_bias, k_alpha, k_gamma, k_beta)
    pv = _pack(v_W, v_bias, v_alpha, v_gamma, v_beta)
    W_cat, b_cat, a_cat, g_cat, be_cat = (
        jnp.concatenate([pq[i], pk[i], pv[i]], axis=1) for i in range(5))

    sizes = [E] * H + [E] * H + [Dh] * H
    gid = np.repeat(np.arange(NG), sizes)
    M = jnp.asarray((gid[:, None] == np.arange(NG)[None, :]).astype(np.float32))
    Mt = M.T
    invcnt = jnp.asarray(1.0 / (F * np.asarray(sizes, np.float32)))[None, :]
    St = jnp.asarray((np.arange(T)[:, None] ==
                      (np.arange(T * F) // F)[None, :]).astype(np.float32)
                     ).astype(jnp.bfloat16)

    x2 = x.reshape(B, D, P)
    kern = functools.partial(_fused_kernel, H=H, E=E, Dh=Dh, T=T, F=F,
                             scale=1.0 / sqrt(F * E))
    out = pl.pallas_call(
        kern,
        out_shape=jax.ShapeDtypeStruct((B, D, P), jnp.float32),
        grid=(B,),
        in_specs=[
            pl.BlockSpec((1, D, P), lambda b: (b, 0, 0)),
            pl.BlockSpec((D, Ctot), lambda b: (0, 0)),
            pl.BlockSpec((1, Ctot), lambda b: (0, 0)),
            pl.BlockSpec((1, Ctot), lambda b: (0, 0)),
            pl.BlockSpec((F, Ctot), lambda b: (0, 0)),
            pl.BlockSpec((F, Ctot), lambda b: (0, 0)),
            pl.BlockSpec((Ctot, NG), lambda b: (0, 0)),
            pl.BlockSpec((NG, Ctot), lambda b: (0, 0)),
            pl.BlockSpec((1, NG), lambda b: (0, 0)),
            pl.BlockSpec((T, P), lambda b: (0, 0)),
            pl.BlockSpec((D, D), lambda b: (0, 0)),
            pl.BlockSpec((1, D), lambda b: (0, 0)),
            pl.BlockSpec((1, D), lambda b: (0, 0)),
            pl.BlockSpec((F, D), lambda b: (0, 0)),
            pl.BlockSpec((F, D), lambda b: (0, 0)),
        ],
        out_specs=pl.BlockSpec((1, D, P), lambda b: (b, 0, 0)),
        compiler_params=pltpu.CompilerParams(
            dimension_semantics=("parallel",),
            vmem_limit_bytes=100 * 1024 * 1024),
    )(x2, W_cat.astype(jnp.bfloat16), b_cat, a_cat, g_cat, be_cat,
      M, Mt, invcnt, St,
      proj_W[0].astype(jnp.bfloat16), proj_bias[0],
      jnp.broadcast_to(proj_alpha[0].reshape(1, 1), (1, D)),
      proj_gamma[0], proj_beta[0])

    return out.reshape(B, D, T, F)


# speedup vs baseline: 1.7591x; 1.0011x over previous
import functools
from math import sqrt

import numpy as np
import jax
import jax.numpy as jnp
from jax import lax
from jax.experimental import pallas as pl
from jax.experimental.pallas import tpu as pltpu

EPS = 1e-5


def _fused_kernel(x_ref, w_ref, b_ref, a_ref, g_ref, be_ref,
                  m_ref, mt_ref, ic_ref, st_ref,
                  wp_ref, bp_ref, ap_ref, gp_ref, bep_ref,
                  o_ref, *, H, E, Dh, T, F, scale):
    D = x_ref.shape[1]
    P = T * F
    EF = E * F
    DhF = Dh * F
    Ctot = w_ref.shape[-1]

    xf = x_ref[0]
    xT = jnp.transpose(xf.astype(jnp.bfloat16))

    y = jnp.dot(xT, w_ref[...], preferred_element_type=jnp.float32)
    y = y + b_ref[...]
    y = jnp.where(y >= 0.0, y, a_ref[...] * y)
    y3 = y.reshape(T, F, Ctot)

    yb = y.astype(jnp.bfloat16)
    s1 = jnp.dot(st_ref[...], yb, preferred_element_type=jnp.float32)
    s2 = jnp.dot(st_ref[...], (yb * yb).astype(jnp.bfloat16),
                 preferred_element_type=jnp.float32)
    mu_g = jnp.dot(s1, m_ref[...],
                   preferred_element_type=jnp.float32) * ic_ref[...]
    sq_g = jnp.dot(s2, m_ref[...],
                   preferred_element_type=jnp.float32) * ic_ref[...]
    inv_g = lax.rsqrt(sq_g - mu_g * mu_g + EPS)
    mu = jnp.dot(mu_g, mt_ref[...], preferred_element_type=jnp.float32)
    inv = jnp.dot(inv_g, mt_ref[...], preferred_element_type=jnp.float32)
    z = (y3 - mu[:, None, :]) * inv[:, None, :] * g_ref[...][None] \
        + be_ref[...][None]

    zs = jnp.transpose(z.astype(jnp.bfloat16), (0, 2, 1))
    HE = H * E
    zzqk = zs[:, :2 * HE, :].reshape(T, 2 * HE * F)
    qoff, koff = 0, H * EF

    ss = []
    for h in range(H):
        qh = zzqk[:, qoff + h * EF:qoff + (h + 1) * EF]
        kh = zzqk[:, koff + h * EF:koff + (h + 1) * EF]
        ss.append(lax.dot_general(qh, kh, (((1,), (1,)), ((), ())),
                                  preferred_element_type=jnp.float32) * scale)
    zzv = zs[:, 2 * HE:, :].reshape(T, H * DhF)
    ps = []
    for h in range(H):
        s = ss[h]
        mx = jnp.max(s, axis=-1, keepdims=True)
        p = jnp.exp(s - mx)
        p = p * (1.0 / jnp.sum(p, axis=-1, keepdims=True))
        ps.append(p.astype(jnp.bfloat16))
    a_parts = []
    for h in range(H):
        vh = zzv[:, h * DhF:(h + 1) * DhF]
        ah = jnp.dot(ps[h], vh, preferred_element_type=jnp.float32)
        a_parts.append(ah.astype(jnp.bfloat16))
    Aall = jnp.concatenate(a_parts, axis=-1)
    At = jnp.transpose(Aall.reshape(T, D, F), (0, 2, 1))
    A2 = At.reshape(P, D)

    o = jnp.dot(A2, wp_ref[...], preferred_element_type=jnp.float32)
    o = o + bp_ref[...]
    o = jnp.where(o >= 0.0, o, ap_ref[...] * o)

    ob = o.astype(jnp.bfloat16)
    t1 = jnp.dot(st_ref[...], ob, preferred_element_type=jnp.float32)
    t2 = jnp.dot(st_ref[...], (ob * ob).astype(jnp.bfloat16),
                 preferred_element_type=jnp.float32)
    cnt = 1.0 / (F * D)
    mu2 = jnp.sum(t1, axis=1, keepdims=True) * cnt
    sq2 = jnp.sum(t2, axis=1, keepdims=True) * cnt
    inv2 = lax.rsqrt(sq2 - mu2 * mu2 + EPS)
    o3 = o.reshape(T, F, D)
    on = (o3 - mu2[:, :, None]) * inv2[:, :, None] * gp_ref[...][None] \
        + bep_ref[...][None]

    o_ref[0] = jnp.transpose(on.reshape(P, D)) + xf


def _pack(W, bias, alpha, gamma, beta):
    G, Cin, Cout = W.shape
    F = gamma.shape[1]
    Wc = jnp.transpose(W, (1, 0, 2)).reshape(Cin, G * Cout)
    bc = jnp.transpose(bias, (1, 0, 2)).reshape(1, G * Cout)
    ac = jnp.repeat(alpha.reshape(G, 1), Cout, axis=1).reshape(1, G * Cout)
    gc = jnp.transpose(gamma, (1, 0, 2)).reshape(F, G * Cout)
    bec = jnp.transpose(beta, (1, 0, 2)).reshape(F, G * Cout)
    return Wc, bc, ac, gc, bec


def kernel(x, q_W, q_bias, q_alpha, q_gamma, q_beta,
           k_W, k_bias, k_alpha, k_gamma, k_beta,
           v_W, v_bias, v_alpha, v_gamma, v_beta,
           proj_W, proj_bias, proj_alpha, proj_gamma, proj_beta):
    B, D, T, F = x.shape
    H, _, E = q_W.shape
    Dh = D // H
    P = T * F
    Ctot = 2 * H * E + H * Dh
    NG = 3 * H

    pq = _pack(q_W, q_bias, q_alpha, q_gamma, q_beta)
    pk = _pack(k_W, k_bias, k_alpha, k_gamma, k_beta)
    pv = _pack(v_W, v_bias, v_alpha, v_gamma, v_beta)
    W_cat, b_cat, a_cat, g_cat, be_cat = (
        jnp.concatenate([pq[i], pk[i], pv[i]], axis=1) for i in range(5))

    sizes = [E] * H + [E] * H + [Dh] * H
    gid = np.repeat(np.arange(NG), sizes)
    M = jnp.asarray((gid[:, None] == np.arange(NG)[None, :]).astype(np.float32))
    Mt = M.T
    invcnt = jnp.asarray(1.0 / (F * np.asarray(sizes, np.float32)))[None, :]
    St = jnp.asarray((np.arange(T)[:, None] ==
                      (np.arange(T * F) // F)[None, :]).astype(np.float32)
                     ).astype(jnp.bfloat16)

    x2 = x.reshape(B, D, P)
    kern = functools.partial(_fused_kernel, H=H, E=E, Dh=Dh, T=T, F=F,
                             scale=1.0 / sqrt(F * E))
    out = pl.pallas_call(
        kern,
        out_shape=jax.ShapeDtypeStruct((B, D, P), jnp.float32),
        grid=(B,),
        in_specs=[
            pl.BlockSpec((1, D, P), lambda b: (b, 0, 0)),
            pl.BlockSpec((D, Ctot), lambda b: (0, 0)),
            pl.BlockSpec((1, Ctot), lambda b: (0, 0)),
            pl.BlockSpec((1, Ctot), lambda b: (0, 0)),
            pl.BlockSpec((F, Ctot), lambda b: (0, 0)),
            pl.BlockSpec((F, Ctot), lambda b: (0, 0)),
            pl.BlockSpec((Ctot, NG), lambda b: (0, 0)),
            pl.BlockSpec((NG, Ctot), lambda b: (0, 0)),
            pl.BlockSpec((1, NG), lambda b: (0, 0)),
            pl.BlockSpec((T, P), lambda b: (0, 0)),
            pl.BlockSpec((D, D), lambda b: (0, 0)),
            pl.BlockSpec((1, D), lambda b: (0, 0)),
            pl.BlockSpec((1, D), lambda b: (0, 0)),
            pl.BlockSpec((F, D), lambda b: (0, 0)),
            pl.BlockSpec((F, D), lambda b: (0, 0)),
        ],
        out_specs=pl.BlockSpec((1, D, P), lambda b: (b, 0, 0)),
        compiler_params=pltpu.CompilerParams(
            dimension_semantics=("parallel",),
            vmem_limit_bytes=100 * 1024 * 1024),
    )(x2, W_cat.astype(jnp.bfloat16), b_cat, a_cat, g_cat, be_cat,
      M, Mt, invcnt, St,
      proj_W[0].astype(jnp.bfloat16), proj_bias[0],
      jnp.broadcast_to(proj_alpha[0].reshape(1, 1), (1, D)),
      proj_gamma[0], proj_beta[0])

    return out.reshape(B, D, T, F)


# speedup vs baseline: 2.3629x; 1.3432x over previous
import functools
from math import sqrt

import numpy as np
import jax
import jax.numpy as jnp
from jax import lax
from jax.experimental import pallas as pl
from jax.experimental.pallas import tpu as pltpu

EPS = 1e-5


def _fused_kernel(x_ref, w_ref, b_ref, a_ref, g_ref, be_ref,
                  m_ref, mt_ref, ic_ref, st_ref,
                  wp_ref, bp_ref, ap_ref, gp_ref, bep_ref,
                  o_ref, *, H, E, Dh, T, F, scale):
    D = x_ref.shape[-1]
    P = T * F
    EF = E * F
    DhF = Dh * F
    Ctot = w_ref.shape[-1]

    xP = x_ref[0].reshape(P, D)

    y = jnp.dot(xP.astype(jnp.bfloat16), w_ref[...],
                preferred_element_type=jnp.float32)
    y = y + b_ref[...]
    y = jnp.where(y >= 0.0, y, a_ref[...] * y)
    y3 = y.reshape(T, F, Ctot)

    yb = y.astype(jnp.bfloat16)
    s1 = jnp.dot(st_ref[...], yb, preferred_element_type=jnp.float32)
    s2 = jnp.dot(st_ref[...], (yb * yb).astype(jnp.bfloat16),
                 preferred_element_type=jnp.float32)
    mu_g = jnp.dot(s1, m_ref[...],
                   preferred_element_type=jnp.float32) * ic_ref[...]
    sq_g = jnp.dot(s2, m_ref[...],
                   preferred_element_type=jnp.float32) * ic_ref[...]
    inv_g = lax.rsqrt(sq_g - mu_g * mu_g + EPS)
    mu = jnp.dot(mu_g, mt_ref[...], preferred_element_type=jnp.float32)
    inv = jnp.dot(inv_g, mt_ref[...], preferred_element_type=jnp.float32)
    z = (y3 - mu[:, None, :]) * inv[:, None, :] * g_ref[...][None] \
        + be_ref[...][None]

    zs = jnp.transpose(z.astype(jnp.bfloat16), (0, 2, 1))
    HE = H * E
    zzqk = zs[:, :2 * HE, :].reshape(T, 2 * HE * F)
    qoff, koff = 0, H * EF

    ss = []
    for h in range(H):
        qh = zzqk[:, qoff + h * EF:qoff + (h + 1) * EF]
        kh = zzqk[:, koff + h * EF:koff + (h + 1) * EF]
        ss.append(lax.dot_general(qh, kh, (((1,), (1,)), ((), ())),
                                  preferred_element_type=jnp.float32) * scale)
    zzv = zs[:, 2 * HE:, :].reshape(T, H * DhF)
    ps = []
    for h in range(H):
        s = ss[h]
        mx = jnp.max(s, axis=-1, keepdims=True)
        p = jnp.exp(s - mx)
        p = p * (1.0 / jnp.sum(p, axis=-1, keepdims=True))
        ps.append(p.astype(jnp.bfloat16))
    a_parts = []
    for h in range(H):
        vh = zzv[:, h * DhF:(h + 1) * DhF]
        ah = jnp.dot(ps[h], vh, preferred_element_type=jnp.float32)
        a_parts.append(ah.astype(jnp.bfloat16))
    Aall = jnp.concatenate(a_parts, axis=-1)
    At = jnp.transpose(Aall.reshape(T, D, F), (0, 2, 1))
    A2 = At.reshape(P, D)

    o = jnp.dot(A2, wp_ref[...], preferred_element_type=jnp.float32)
    o = o + bp_ref[...]
    o = jnp.where(o >= 0.0, o, ap_ref[...] * o)

    ob = o.astype(jnp.bfloat16)
    t1 = jnp.dot(st_ref[...], ob, preferred_element_type=jnp.float32)
    t2 = jnp.dot(st_ref[...], (ob * ob).astype(jnp.bfloat16),
                 preferred_element_type=jnp.float32)
    cnt = 1.0 / (F * D)
    mu2 = jnp.sum(t1, axis=1, keepdims=True) * cnt
    sq2 = jnp.sum(t2, axis=1, keepdims=True) * cnt
    inv2 = lax.rsqrt(sq2 - mu2 * mu2 + EPS)
    o3 = o.reshape(T, F, D)
    on = (o3 - mu2[:, :, None]) * inv2[:, :, None] * gp_ref[...][None] \
        + bep_ref[...][None]

    o_ref[0] = on + x_ref[0]


def _pack(W, bias, alpha, gamma, beta):
    G, Cin, Cout = W.shape
    F = gamma.shape[1]
    Wc = jnp.transpose(W, (1, 0, 2)).reshape(Cin, G * Cout)
    bc = jnp.transpose(bias, (1, 0, 2)).reshape(1, G * Cout)
    ac = jnp.repeat(alpha.reshape(G, 1), Cout, axis=1).reshape(1, G * Cout)
    gc = jnp.transpose(gamma, (1, 0, 2)).reshape(F, G * Cout)
    bec = jnp.transpose(beta, (1, 0, 2)).reshape(F, G * Cout)
    return Wc, bc, ac, gc, bec


def kernel(x, q_W, q_bias, q_alpha, q_gamma, q_beta,
           k_W, k_bias, k_alpha, k_gamma, k_beta,
           v_W, v_bias, v_alpha, v_gamma, v_beta,
           proj_W, proj_bias, proj_alpha, proj_gamma, proj_beta):
    B, D, T, F = x.shape
    H, _, E = q_W.shape
    Dh = D // H
    P = T * F
    Ctot = 2 * H * E + H * Dh
    NG = 3 * H

    pq = _pack(q_W, q_bias, q_alpha, q_gamma, q_beta)
    pk = _pack(k_W, k_bias, k_alpha, k_gamma, k_beta)
    pv = _pack(v_W, v_bias, v_alpha, v_gamma, v_beta)
    W_cat, b_cat, a_cat, g_cat, be_cat = (
        jnp.concatenate([pq[i], pk[i], pv[i]], axis=1) for i in range(5))

    sizes = [E] * H + [E] * H + [Dh] * H
    gid = np.repeat(np.arange(NG), sizes)
    M = jnp.asarray((gid[:, None] == np.arange(NG)[None, :]).astype(np.float32))
    Mt = M.T
    invcnt = jnp.asarray(1.0 / (F * np.asarray(sizes, np.float32)))[None, :]
    St = jnp.asarray((np.arange(T)[:, None] ==
                      (np.arange(T * F) // F)[None, :]).astype(np.float32)
                     ).astype(jnp.bfloat16)

    x_cl = jnp.transpose(x, (0, 2, 3, 1))
    kern = functools.partial(_fused_kernel, H=H, E=E, Dh=Dh, T=T, F=F,
                             scale=1.0 / sqrt(F * E))
    out = pl.pallas_call(
        kern,
        out_shape=jax.ShapeDtypeStruct((B, T, F, D), jnp.float32),
        grid=(B,),
        in_specs=[
            pl.BlockSpec((1, T, F, D), lambda b: (b, 0, 0, 0)),
            pl.BlockSpec((D, Ctot), lambda b: (0, 0)),
            pl.BlockSpec((1, Ctot), lambda b: (0, 0)),
            pl.BlockSpec((1, Ctot), lambda b: (0, 0)),
            pl.BlockSpec((F, Ctot), lambda b: (0, 0)),
            pl.BlockSpec((F, Ctot), lambda b: (0, 0)),
            pl.BlockSpec((Ctot, NG), lambda b: (0, 0)),
            pl.BlockSpec((NG, Ctot), lambda b: (0, 0)),
            pl.BlockSpec((1, NG), lambda b: (0, 0)),
            pl.BlockSpec((T, P), lambda b: (0, 0)),
            pl.BlockSpec((D, D), lambda b: (0, 0)),
            pl.BlockSpec((1, D), lambda b: (0, 0)),
            pl.BlockSpec((1, D), lambda b: (0, 0)),
            pl.BlockSpec((F, D), lambda b: (0, 0)),
            pl.BlockSpec((F, D), lambda b: (0, 0)),
        ],
        out_specs=pl.BlockSpec((1, T, F, D), lambda b: (b, 0, 0, 0)),
        compiler_params=pltpu.CompilerParams(
            dimension_semantics=("parallel",),
            vmem_limit_bytes=100 * 1024 * 1024),
    )(x_cl, W_cat.astype(jnp.bfloat16), b_cat, a_cat, g_cat, be_cat,
      M, Mt, invcnt, St,
      proj_W[0].astype(jnp.bfloat16), proj_bias[0],
      jnp.broadcast_to(proj_alpha[0].reshape(1, 1), (1, D)),
      proj_gamma[0], proj_beta[0])

    return jnp.transpose(out, (0, 3, 1, 2))


# speedup vs baseline: 2.5028x; 1.0592x over previous
import functools
from math import sqrt

import numpy as np
import jax
import jax.numpy as jnp
from jax import lax
from jax.experimental import pallas as pl
from jax.experimental.pallas import tpu as pltpu

EPS = 1e-5


def _fused_kernel(x_ref, w_ref, b_ref, a_ref, g_ref, be_ref,
                  m_ref, mt_ref, ic_ref, st_ref,
                  wp_ref, bp_ref, ap_ref, gp_ref, bep_ref,
                  o_ref, *, H, E, Dh, T, F, scale):
    D = x_ref.shape[-1]
    P = T * F
    EF = E * F
    DhF = Dh * F
    Ctot = w_ref.shape[-1]

    xP = x_ref[0].reshape(P, D)

    y = jnp.dot(xP.astype(jnp.bfloat16), w_ref[...],
                preferred_element_type=jnp.float32)
    y = y + b_ref[...]
    yb = y.astype(jnp.bfloat16)
    yb = jnp.where(yb >= 0, yb, a_ref[...] * yb)

    s1 = jnp.dot(st_ref[...], yb, preferred_element_type=jnp.float32)
    s2 = jnp.dot(st_ref[...], yb * yb,
                 preferred_element_type=jnp.float32)
    mu_g = jnp.dot(s1, m_ref[...],
                   preferred_element_type=jnp.float32) * ic_ref[...]
    sq_g = jnp.dot(s2, m_ref[...],
                   preferred_element_type=jnp.float32) * ic_ref[...]
    inv_g = lax.rsqrt(sq_g - mu_g * mu_g + EPS)
    mu = jnp.dot(mu_g, mt_ref[...], preferred_element_type=jnp.float32)
    inv = jnp.dot(inv_g, mt_ref[...], preferred_element_type=jnp.float32)
    y3b = yb.reshape(T, F, Ctot)
    z = (y3b - mu.astype(jnp.bfloat16)[:, None, :]) \
        * inv.astype(jnp.bfloat16)[:, None, :] * g_ref[...][None] \
        + be_ref[...][None]

    zs = jnp.transpose(z, (0, 2, 1))
    HE = H * E
    zzqk = zs[:, :2 * HE, :].reshape(T, 2 * HE * F)
    qoff, koff = 0, H * EF

    ss = []
    for h in range(H):
        qh = zzqk[:, qoff + h * EF:qoff + (h + 1) * EF]
        kh = zzqk[:, koff + h * EF:koff + (h + 1) * EF]
        ss.append(lax.dot_general(qh, kh, (((1,), (1,)), ((), ())),
                                  preferred_element_type=jnp.float32) * scale)
    zzv = zs[:, 2 * HE:, :].reshape(T, H * DhF)
    ps = []
    for h in range(H):
        s = ss[h]
        mx = jnp.max(s, axis=-1, keepdims=True)
        p = jnp.exp(s - mx)
        p = p * (1.0 / jnp.sum(p, axis=-1, keepdims=True))
        ps.append(p.astype(jnp.bfloat16))
    a_parts = []
    for h in range(H):
        vh = zzv[:, h * DhF:(h + 1) * DhF]
        ah = jnp.dot(ps[h], vh, preferred_element_type=jnp.float32)
        a_parts.append(ah.astype(jnp.bfloat16))
    Aall = jnp.concatenate(a_parts, axis=-1)
    At = jnp.transpose(Aall.reshape(T, D, F), (0, 2, 1))
    A2 = At.reshape(P, D)

    o = jnp.dot(A2, wp_ref[...], preferred_element_type=jnp.float32)
    o = o + bp_ref[...]
    ob = o.astype(jnp.bfloat16)
    ob = jnp.where(ob >= 0, ob, ap_ref[...] * ob)

    t1 = jnp.dot(st_ref[...], ob, preferred_element_type=jnp.float32)
    t2 = jnp.dot(st_ref[...], ob * ob,
                 preferred_element_type=jnp.float32)
    cnt = 1.0 / (F * D)
    mu2 = jnp.sum(t1, axis=1, keepdims=True) * cnt
    sq2 = jnp.sum(t2, axis=1, keepdims=True) * cnt
    inv2 = lax.rsqrt(sq2 - mu2 * mu2 + EPS)
    o3b = ob.reshape(T, F, D)
    on = (o3b - mu2.astype(jnp.bfloat16)[:, :, None]) \
        * inv2.astype(jnp.bfloat16)[:, :, None] * gp_ref[...][None] \
        + bep_ref[...][None]

    o_ref[0] = on.astype(jnp.float32) + x_ref[0]


def _pack(W, bias, alpha, gamma, beta):
    G, Cin, Cout = W.shape
    F = gamma.shape[1]
    Wc = jnp.transpose(W, (1, 0, 2)).reshape(Cin, G * Cout)
    bc = jnp.transpose(bias, (1, 0, 2)).reshape(1, G * Cout)
    ac = jnp.repeat(alpha.reshape(G, 1), Cout, axis=1).reshape(1, G * Cout)
    gc = jnp.transpose(gamma, (1, 0, 2)).reshape(F, G * Cout)
    bec = jnp.transpose(beta, (1, 0, 2)).reshape(F, G * Cout)
    return Wc, bc, ac, gc, bec


def kernel(x, q_W, q_bias, q_alpha, q_gamma, q_beta,
           k_W, k_bias, k_alpha, k_gamma, k_beta,
           v_W, v_bias, v_alpha, v_gamma, v_beta,
           proj_W, proj_bias, proj_alpha, proj_gamma, proj_beta):
    B, D, T, F = x.shape
    H, _, E = q_W.shape
    Dh = D // H
    P = T * F
    Ctot = 2 * H * E + H * Dh
    NG = 3 * H

    pq = _pack(q_W, q_bias, q_alpha, q_gamma, q_beta)
    pk = _pack(k_W, k_bias, k_alpha, k_gamma, k_beta)
    pv = _pack(v_W, v_bias, v_alpha, v_gamma, v_beta)
    W_cat, b_cat, a_cat, g_cat, be_cat = (
        jnp.concatenate([pq[i], pk[i], pv[i]], axis=1) for i in range(5))

    sizes = [E] * H + [E] * H + [Dh] * H
    gid = np.repeat(np.arange(NG), sizes)
    M = jnp.asarray((gid[:, None] == np.arange(NG)[None, :]).astype(np.float32))
    Mt = M.T
    invcnt = jnp.asarray(1.0 / (F * np.asarray(sizes, np.float32)))[None, :]
    St = jnp.asarray((np.arange(T)[:, None] ==
                      (np.arange(T * F) // F)[None, :]).astype(np.float32)
                     ).astype(jnp.bfloat16)

    x_cl = jnp.transpose(x, (0, 2, 3, 1))
    kern = functools.partial(_fused_kernel, H=H, E=E, Dh=Dh, T=T, F=F,
                             scale=1.0 / sqrt(F * E))
    out = pl.pallas_call(
        kern,
        out_shape=jax.ShapeDtypeStruct((B, T, F, D), jnp.float32),
        grid=(B,),
        in_specs=[
            pl.BlockSpec((1, T, F, D), lambda b: (b, 0, 0, 0)),
            pl.BlockSpec((D, Ctot), lambda b: (0, 0)),
            pl.BlockSpec((1, Ctot), lambda b: (0, 0)),
            pl.BlockSpec((1, Ctot), lambda b: (0, 0)),
            pl.BlockSpec((F, Ctot), lambda b: (0, 0)),
            pl.BlockSpec((F, Ctot), lambda b: (0, 0)),
            pl.BlockSpec((Ctot, NG), lambda b: (0, 0)),
            pl.BlockSpec((NG, Ctot), lambda b: (0, 0)),
            pl.BlockSpec((1, NG), lambda b: (0, 0)),
            pl.BlockSpec((T, P), lambda b: (0, 0)),
            pl.BlockSpec((D, D), lambda b: (0, 0)),
            pl.BlockSpec((1, D), lambda b: (0, 0)),
            pl.BlockSpec((1, D), lambda b: (0, 0)),
            pl.BlockSpec((F, D), lambda b: (0, 0)),
            pl.BlockSpec((F, D), lambda b: (0, 0)),
        ],
        out_specs=pl.BlockSpec((1, T, F, D), lambda b: (b, 0, 0, 0)),
        compiler_params=pltpu.CompilerParams(
            dimension_semantics=("parallel",),
            vmem_limit_bytes=100 * 1024 * 1024),
    )(x_cl, W_cat.astype(jnp.bfloat16), b_cat,
      a_cat.astype(jnp.bfloat16), g_cat.astype(jnp.bfloat16),
      be_cat.astype(jnp.bfloat16),
      M, Mt, invcnt, St,
      proj_W[0].astype(jnp.bfloat16), proj_bias[0],
      jnp.broadcast_to(proj_alpha[0].reshape(1, 1), (1, D)).astype(jnp.bfloat16),
      proj_gamma[0].astype(jnp.bfloat16), proj_beta[0].astype(jnp.bfloat16))

    return jnp.transpose(out, (0, 3, 1, 2))
